# baseline (speedup 1.0000x reference)
import functools

import numpy as np

import jax
import jax.numpy as jnp
from jax import lax
from jax.experimental import pallas as pl
from jax.experimental.pallas import tpu as pltpu
from jax.experimental.pallas import tpu_sc as plsc

NC = 2
NS = 16
LANES = 16
CH = 128
W128 = 128
ZR = 64
NBUF = 3


def _splits(n):
  half = n // NC
  acc_rows = half + 8
  rows_lo = (half // (8 * NS)) * 8
  rows_last = half - (NS - 1) * rows_lo
  assert half % 8 == 0 and rows_last % 8 == 0 and rows_last > 0
  return half, acc_rows, rows_lo, rows_last


def _remap_dst(ed_v, dst_v, c_off, half, spread):
  for j in range(CH // LANES):
    loc = ed_v[pl.ds(CH + j * LANES, LANES)] - c_off
    ok = (loc >= 0) & (loc < half)
    trash = jnp.int32(half) + (spread & 7)
    dst_v[pl.ds(j * LANES, LANES)] = jnp.where(ok, loc, trash)


def _copy_out(acc_sh, out_hbm, s, c_off, rows_lo, rows_last):
  @pl.when(s < NS - 1)
  def _():
    r0 = s * rows_lo
    pltpu.sync_copy(acc_sh.at[pl.ds(r0, rows_lo)],
                    out_hbm.at[pl.ds(c_off + r0, rows_lo)])

  @pl.when(s == NS - 1)
  def _():
    r0 = (NS - 1) * rows_lo
    pltpu.sync_copy(acc_sh.at[pl.ds(r0, rows_last)],
                    out_hbm.at[pl.ds(c_off + r0, rows_last)])


def _zero_acc(buf, accs, s, rows_lo, rows_last):
  d = buf.shape[1]

  def zrow(r, carry):
    for j in range(d // LANES):
      buf[r, pl.ds(j * LANES, LANES)] = jnp.zeros((LANES,), jnp.float32)
    return carry
  lax.fori_loop(jnp.int32(0), jnp.int32(ZR), zrow, jnp.int32(0))

  def zspan(r0, nrows):
    off = 0
    while off < nrows:
      step = min(ZR, nrows - off)
      for acc in accs:
        pltpu.sync_copy(buf.at[pl.ds(0, step)], acc.at[pl.ds(r0 + off, step)])
      off += step

  @pl.when(s < NS - 1)
  def _():
    zspan(s * rows_lo, rows_lo)

  @pl.when(s == NS - 1)
  def _():
    zspan((NS - 1) * rows_lo, rows_last + 8)


def _seg_sum_builder(n, n_blocks):
  acc_rows = n + 8
  rows_lo = (n // (8 * NS)) * 8
  rows_last = n - (NS - 1) * rows_lo
  assert n % 8 == 0 and rows_last % 8 == 0 and rows_last > 0
  bpt = n_blocks // NS
  assert bpt % NBUF == 0
  outer = bpt // NBUF

  mesh = plsc.VectorSubcoreMesh(core_axis_name="c", subcore_axis_name="s")

  @functools.partial(
      pl.kernel,
      out_type=(jax.ShapeDtypeStruct((n, W128), jnp.float32),
                jax.ShapeDtypeStruct((n, W128), jnp.float32)),
      mesh=mesh,
      scratch_types=(
          [pltpu.VMEM((2 * CH,), jnp.int32) for _ in range(NBUF)]
          + [pltpu.VMEM((CH,), jnp.int32) for _ in range(NBUF)]
          + [pltpu.VMEM((CH, W128), jnp.float32) for _ in range(NBUF)]
          + [pltpu.VMEM_SHARED((acc_rows, W128), jnp.float32)]
          + [pltpu.SemaphoreType.DMA for _ in range(3 * NBUF)]
      ),
  )
  def seg_sum(x_lo_hbm, x_hi_hbm, edges_hbm, out_lo_hbm, out_hi_hbm, *scr):
    ed = scr[0:NBUF]
    dst = scr[NBUF:2 * NBUF]
    ra = scr[2 * NBUF:3 * NBUF]
    acc = scr[3 * NBUF]
    isem = scr[3 * NBUF + 1:3 * NBUF + 1 + NBUF]
    gsem = scr[3 * NBUF + 1 + NBUF:3 * NBUF + 1 + 2 * NBUF]
    ssem = scr[3 * NBUF + 1 + 2 * NBUF:3 * NBUF + 1 + 3 * NBUF]

    c = lax.axis_index("c")
    s = lax.axis_index("s")

    _zero_acc(ra[0], (acc,), s, rows_lo, rows_last)
    plsc.subcore_barrier()

    base_blk = s * jnp.int32(bpt)

    def copy_dst(b):
      for j in range(CH // LANES):
        dst[b][pl.ds(j * LANES, LANES)] = ed[b][pl.ds(CH + j * LANES, LANES)]

    def gather(b):
      src = ed[b].at[pl.ds(0, CH)]

      @pl.when(c == 0)
      def _():
        pltpu.async_copy(x_lo_hbm.at[src], ra[b], gsem[b])

      @pl.when(c == 1)
      def _():
        pltpu.async_copy(x_hi_hbm.at[src], ra[b], gsem[b])

    def wait_gather(b):
      src = ed[b].at[pl.ds(0, CH)]
      pltpu.make_async_copy(x_lo_hbm.at[src], ra[b], gsem[b]).wait()

    def scatter(b):
      pltpu.async_copy(ra[b], acc.at[dst[b]], ssem[b], add=True)

    def wait_scatter(b):
      pltpu.make_async_copy(ra[b], acc.at[dst[b]], ssem[b]).wait()

    def wait_idx(b):
      pltpu.make_async_copy(edges_hbm.at[base_blk], ed[b], isem[b]).wait()

    pltpu.async_copy(edges_hbm.at[base_blk], ed[0], isem[0])

    def body(g, carry):
      for b in range(NBUF):
        i = NBUF * g + jnp.int32(b)
        pb = (b - 1) % NBUF
        nb = (b + 1) % NBUF

        @pl.when(g > 0)
        def _():
          wait_scatter(b)
        wait_idx(b)
        copy_dst(b)
        gather(b)
        if b == 0:
          @pl.when(g > 0)
          def _():
            wait_gather(pb)
            scatter(pb)
        else:
          wait_gather(pb)
          scatter(pb)
        if b == NBUF - 1:
          @pl.when(g < outer - 1)
          def _():
            pltpu.async_copy(edges_hbm.at[base_blk + i + 1], ed[nb], isem[nb])
        else:
          pltpu.async_copy(edges_hbm.at[base_blk + i + 1], ed[nb], isem[nb])
      return carry

    lax.fori_loop(jnp.int32(0), jnp.int32(outer), body, jnp.int32(0))

    last = NBUF - 1
    wait_gather(last)
    scatter(last)
    for b in range(NBUF):
      wait_scatter(b)

    plsc.subcore_barrier()

    @pl.when(c == 0)
    def _():
      _copy_out(acc, out_lo_hbm, s, jnp.int32(0), rows_lo, rows_last)

    @pl.when(c == 1)
    def _():
      _copy_out(acc, out_hi_hbm, s, jnp.int32(0), rows_lo, rows_last)

  return seg_sum


def _deg_builder(n, n_blocks):
  half, acc_rows, _, _ = _splits(n)
  bpt = n_blocks // NS
  assert bpt % NBUF == 0
  outer = bpt // NBUF
  h640 = acc_rows * LANES // W128
  out_rows = half * LANES // W128
  assert half * LANES % W128 == 0

  mesh = plsc.VectorSubcoreMesh(core_axis_name="c", subcore_axis_name="s")

  @functools.partial(
      pl.kernel,
      out_type=jax.ShapeDtypeStruct((NC, NS, out_rows * W128), jnp.float32),
      mesh=mesh,
      compiler_params=pltpu.CompilerParams(needs_layout_passes=False),
      scratch_types=(
          [pltpu.VMEM((2 * CH,), jnp.int32) for _ in range(NBUF)]
          + [pltpu.VMEM((h640 * W128,), jnp.float32)]
          + [pltpu.SemaphoreType.DMA for _ in range(NBUF)]
      ),
  )
  def degk(edges_hbm, out_hbm, *scr):
    ed = scr[0:NBUF]
    hist = scr[NBUF]
    isem = scr[NBUF + 1:NBUF + 1 + NBUF]

    c = lax.axis_index("c")
    s = lax.axis_index("s")
    c_off = c * jnp.int32(half)

    def zrow(r, carry):
      for j in range(8):
        hist[pl.ds(r * (8 * LANES) + j * LANES, LANES)] = jnp.zeros(
            (LANES,), jnp.float32)
      return carry
    lax.fori_loop(jnp.int32(0), jnp.int32(h640 * W128 // (8 * LANES)), zrow,
                  jnp.int32(0))

    base_blk = s * jnp.int32(bpt)
    spread = lax.iota(jnp.int32, LANES)
    ones16 = jnp.full((LANES,), 1.0, jnp.float32)

    def wait_idx(b):
      pltpu.make_async_copy(edges_hbm.at[base_blk], ed[b], isem[b]).wait()

    pltpu.async_copy(edges_hbm.at[base_blk], ed[0], isem[0])

    def body(g, carry):
      for b in range(NBUF):
        i = NBUF * g + jnp.int32(b)
        nb = (b + 1) % NBUF

        wait_idx(b)
        for j in range(CH // LANES):
          loc = ed[b][pl.ds(CH + j * LANES, LANES)] - c_off
          ok = (loc >= 0) & (loc < half)
          trash = jnp.int32(half) + (spread & 7)
          loc = jnp.where(ok, loc, trash)
          flat = loc * LANES + spread
          plsc.addupdate_scatter(hist, (flat,), ones16)
        if b == NBUF - 1:
          @pl.when(g < outer - 1)
          def _():
            pltpu.async_copy(edges_hbm.at[base_blk + i + 1], ed[nb], isem[nb])
        else:
          pltpu.async_copy(edges_hbm.at[base_blk + i + 1], ed[nb], isem[nb])
      return carry

    lax.fori_loop(jnp.int32(0), jnp.int32(outer), body, jnp.int32(0))
    pltpu.sync_copy(hist.at[pl.ds(0, out_rows * W128)], out_hbm.at[c, s])

  return degk


def _tc_layer(n, d, blk, residual):
  grid = (n // blk,)
  half = n // NC
  blocks_per_core = half // blk
  assert half % blk == 0

  def body(x_ref, alo_ref, ahi_ref, dg_ref, ws_ref, wlo_ref, whi_ref, b_ref,
           o_ref, olo_ref, ohi_ref):
    x = x_ref[...]
    deg = jnp.sum(dg_ref[0], axis=(0, 2))[:, None]
    invd = 1.0 / jnp.maximum(deg, 1.0)
    h = (
        jnp.dot(x, ws_ref[...], preferred_element_type=jnp.float32)
        + jnp.dot(alo_ref[...] * invd, wlo_ref[...],
                  preferred_element_type=jnp.float32)
        + jnp.dot(ahi_ref[...] * invd, whi_ref[...],
                  preferred_element_type=jnp.float32)
        + b_ref[...]
    )
    h = jnp.maximum(h, 0.0)
    o_ref[...] = x + h if residual else h
    olo_ref[...] = h[:, :W128]
    ohi_ref[...] = h[:, W128:]

  z = np.int32(0)
  row_map = lambda i: (i, z)
  const_map = lambda i: (z, z)
  deg_map = lambda i: (i // blocks_per_core, z, i % blocks_per_core, z)
  return pl.pallas_call(
      body,
      grid=grid,
      in_specs=[
          pl.BlockSpec((blk, d), row_map),
          pl.BlockSpec((blk, W128), row_map),
          pl.BlockSpec((blk, W128), row_map),
          pl.BlockSpec((1, NS, blk, LANES), deg_map),
          pl.BlockSpec((d, d), const_map),
          pl.BlockSpec((W128, d), const_map),
          pl.BlockSpec((W128, d), const_map),
          pl.BlockSpec((1, d), const_map),
      ],
      out_specs=(pl.BlockSpec((blk, d), row_map),
                 pl.BlockSpec((blk, W128), row_map),
                 pl.BlockSpec((blk, W128), row_map)),
      out_shape=(jax.ShapeDtypeStruct((n, d), jnp.float32),
                 jax.ShapeDtypeStruct((n, W128), jnp.float32),
                 jax.ShapeDtypeStruct((n, W128), jnp.float32)),
  )


@jax.jit
def kernel(X, edge_index, W1_self, W1_neigh, b1, W2_self, W2_neigh, b2):
  n, d = X.shape
  e = edge_index.shape[1]

  src = edge_index[0].astype(jnp.int32)
  dst = edge_index[1].astype(jnp.int32)
  epc = NS * CH * NBUF
  e_pad = ((e + epc - 1) // epc) * epc
  pad = e_pad - e
  if pad:
    arp = jnp.arange(pad, dtype=jnp.int32)
    src = jnp.concatenate([src, (arp * 37) % n])
    dst = jnp.concatenate([dst, n + (arp & 7)])
  n_blocks = e_pad // CH
  edges = (jnp.stack([src, dst]).reshape(2, n_blocks, CH)
           .transpose(1, 0, 2).reshape(n_blocks, 2 * CH))

  seg = _seg_sum_builder(n, n_blocks)
  degk = _deg_builder(n, n_blocks)
  tc1 = _tc_layer(n, d, 1000, residual=False)
  tc2 = _tc_layer(n, d, 1000, residual=True)

  f32 = jnp.float32
  w1s, w1n = W1_self.astype(f32), W1_neigh.astype(f32)
  w2s, w2n = W2_self.astype(f32), W2_neigh.astype(f32)
  bb1 = b1.astype(f32).reshape(1, d)
  bb2 = b2.astype(f32).reshape(1, d)
  w1n_lo = lax.slice(w1n, (0, 0), (W128, d))
  w1n_hi = lax.slice(w1n, (W128, 0), (d, d))
  w2n_lo = lax.slice(w2n, (0, 0), (W128, d))
  w2n_hi = lax.slice(w2n, (W128, 0), (d, d))

  Xf = X.astype(f32)
  x_lo = lax.slice(Xf, (0, 0), (n, W128))
  x_hi = lax.slice(Xf, (0, W128), (n, d))
  half = n // NC
  deg = degk(edges).reshape(NC, NS, half, LANES)
  a1_lo, a1_hi = seg(x_lo, x_hi, edges)
  X1, x1_lo, x1_hi = tc1(Xf, a1_lo, a1_hi, deg, w1s, w1n_lo, w1n_hi, bb1)
  a2_lo, a2_hi = seg(x1_lo, x1_hi, edges)
  out, _, _ = tc2(X1, a2_lo, a2_hi, deg, w2s, w2n_lo, w2n_hi, bb2)
  out_dtype = jnp.result_type(X.dtype, W1_self.dtype)
  return out.astype(out_dtype)


# speedup vs baseline: 1.0090x; 1.0090x over previous
import functools

import numpy as np

import jax
import jax.numpy as jnp
from jax import lax
from jax.experimental import pallas as pl
from jax.experimental.pallas import tpu as pltpu
from jax.experimental.pallas import tpu_sc as plsc

NC = 2
NS = 16
LANES = 16
CH = 128
W128 = 128
ZR = 64
NBUF = 2


def _splits(n):
  half = n // NC
  acc_rows = half + 8
  rows_lo = (half // (8 * NS)) * 8
  rows_last = half - (NS - 1) * rows_lo
  assert half % 8 == 0 and rows_last % 8 == 0 and rows_last > 0
  return half, acc_rows, rows_lo, rows_last


def _remap_dst(ed_v, dst_v, c_off, half, spread):
  for j in range(CH // LANES):
    loc = ed_v[pl.ds(CH + j * LANES, LANES)] - c_off
    ok = (loc >= 0) & (loc < half)
    trash = jnp.int32(half) + (spread & 7)
    dst_v[pl.ds(j * LANES, LANES)] = jnp.where(ok, loc, trash)


def _copy_out(acc_sh, out_hbm, s, c_off, rows_lo, rows_last):
  @pl.when(s < NS - 1)
  def _():
    r0 = s * rows_lo
    pltpu.sync_copy(acc_sh.at[pl.ds(r0, rows_lo)],
                    out_hbm.at[pl.ds(c_off + r0, rows_lo)])

  @pl.when(s == NS - 1)
  def _():
    r0 = (NS - 1) * rows_lo
    pltpu.sync_copy(acc_sh.at[pl.ds(r0, rows_last)],
                    out_hbm.at[pl.ds(c_off + r0, rows_last)])


def _zero_acc(buf, accs, s, rows_lo, rows_last):
  d = buf.shape[1]

  def zrow(r, carry):
    for j in range(d // LANES):
      buf[r, pl.ds(j * LANES, LANES)] = jnp.zeros((LANES,), jnp.float32)
    return carry
  lax.fori_loop(jnp.int32(0), jnp.int32(ZR), zrow, jnp.int32(0))

  def zspan(r0, nrows):
    off = 0
    while off < nrows:
      step = min(ZR, nrows - off)
      for acc in accs:
        pltpu.sync_copy(buf.at[pl.ds(0, step)], acc.at[pl.ds(r0 + off, step)])
      off += step

  @pl.when(s < NS - 1)
  def _():
    zspan(s * rows_lo, rows_lo)

  @pl.when(s == NS - 1)
  def _():
    zspan((NS - 1) * rows_lo, rows_last + 8)


def _seg_sum_builder(n, n_blocks):
  acc_rows = n + 8
  rows_lo = (n // (8 * NS)) * 8
  rows_last = n - (NS - 1) * rows_lo
  assert n % 8 == 0 and rows_last % 8 == 0 and rows_last > 0
  bpt = n_blocks // NS
  assert bpt % NBUF == 0
  outer = bpt // NBUF

  mesh = plsc.VectorSubcoreMesh(core_axis_name="c", subcore_axis_name="s")

  @functools.partial(
      pl.kernel,
      out_type=(jax.ShapeDtypeStruct((n, W128), jnp.float32),
                jax.ShapeDtypeStruct((n, W128), jnp.float32)),
      mesh=mesh,
      scratch_types=(
          [pltpu.VMEM((2 * CH,), jnp.int32) for _ in range(NBUF)]
          + [pltpu.VMEM((CH,), jnp.int32) for _ in range(NBUF)]
          + [pltpu.VMEM((CH, W128), jnp.float32) for _ in range(NBUF)]
          + [pltpu.VMEM_SHARED((acc_rows, W128), jnp.float32)]
          + [pltpu.SemaphoreType.DMA for _ in range(3 * NBUF)]
      ),
  )
  def seg_sum(x_lo_hbm, x_hi_hbm, edges_hbm, out_lo_hbm, out_hi_hbm, *scr):
    ed = scr[0:NBUF]
    dst = scr[NBUF:2 * NBUF]
    ra = scr[2 * NBUF:3 * NBUF]
    acc = scr[3 * NBUF]
    isem = scr[3 * NBUF + 1:3 * NBUF + 1 + NBUF]
    gsem = scr[3 * NBUF + 1 + NBUF:3 * NBUF + 1 + 2 * NBUF]
    ssem = scr[3 * NBUF + 1 + 2 * NBUF:3 * NBUF + 1 + 3 * NBUF]

    c = lax.axis_index("c")
    s = lax.axis_index("s")

    _zero_acc(ra[0], (acc,), s, rows_lo, rows_last)
    plsc.subcore_barrier()

    base_blk = s * jnp.int32(bpt)

    def copy_dst(b):
      for j in range(CH // LANES):
        dst[b][pl.ds(j * LANES, LANES)] = ed[b][pl.ds(CH + j * LANES, LANES)]

    def gather(b):
      src = ed[b].at[pl.ds(0, CH)]

      @pl.when(c == 0)
      def _():
        pltpu.async_copy(x_lo_hbm.at[src], ra[b], gsem[b])

      @pl.when(c == 1)
      def _():
        pltpu.async_copy(x_hi_hbm.at[src], ra[b], gsem[b])

    def wait_gather(b):
      src = ed[b].at[pl.ds(0, CH)]
      pltpu.make_async_copy(x_lo_hbm.at[src], ra[b], gsem[b]).wait()

    def scatter(b):
      pltpu.async_copy(ra[b], acc.at[dst[b]], ssem[b], add=True)

    def wait_scatter(b):
      pltpu.make_async_copy(ra[b], acc.at[dst[b]], ssem[b]).wait()

    def wait_idx(b):
      pltpu.make_async_copy(edges_hbm.at[base_blk], ed[b], isem[b]).wait()

    pltpu.async_copy(edges_hbm.at[base_blk], ed[0], isem[0])

    def body(g, carry):
      for b in range(NBUF):
        i = NBUF * g + jnp.int32(b)
        pb = (b - 1) % NBUF
        nb = (b + 1) % NBUF

        @pl.when(g > 0)
        def _():
          wait_scatter(b)
        wait_idx(b)
        copy_dst(b)
        gather(b)
        if b == 0:
          @pl.when(g > 0)
          def _():
            wait_gather(pb)
            scatter(pb)
        else:
          wait_gather(pb)
          scatter(pb)
        if b == NBUF - 1:
          @pl.when(g < outer - 1)
          def _():
            pltpu.async_copy(edges_hbm.at[base_blk + i + 1], ed[nb], isem[nb])
        else:
          pltpu.async_copy(edges_hbm.at[base_blk + i + 1], ed[nb], isem[nb])
      return carry

    lax.fori_loop(jnp.int32(0), jnp.int32(outer), body, jnp.int32(0))

    last = NBUF - 1
    wait_gather(last)
    scatter(last)
    for b in range(NBUF):
      wait_scatter(b)

    plsc.subcore_barrier()

    @pl.when(c == 0)
    def _():
      _copy_out(acc, out_lo_hbm, s, jnp.int32(0), rows_lo, rows_last)

    @pl.when(c == 1)
    def _():
      _copy_out(acc, out_hi_hbm, s, jnp.int32(0), rows_lo, rows_last)

  return seg_sum


def _deg_builder(n, n_blocks):
  half, acc_rows, _, _ = _splits(n)
  bpt = n_blocks // NS
  assert bpt % NBUF == 0
  outer = bpt // NBUF
  h640 = acc_rows * LANES // W128
  out_rows = half * LANES // W128
  assert half * LANES % W128 == 0

  mesh = plsc.VectorSubcoreMesh(core_axis_name="c", subcore_axis_name="s")

  @functools.partial(
      pl.kernel,
      out_type=jax.ShapeDtypeStruct((NC, NS, out_rows * W128), jnp.float32),
      mesh=mesh,
      compiler_params=pltpu.CompilerParams(needs_layout_passes=False),
      scratch_types=(
          [pltpu.VMEM((2 * CH,), jnp.int32) for _ in range(NBUF)]
          + [pltpu.VMEM((h640 * W128,), jnp.float32)]
          + [pltpu.SemaphoreType.DMA for _ in range(NBUF)]
      ),
  )
  def degk(edges_hbm, out_hbm, *scr):
    ed = scr[0:NBUF]
    hist = scr[NBUF]
    isem = scr[NBUF + 1:NBUF + 1 + NBUF]

    c = lax.axis_index("c")
    s = lax.axis_index("s")
    c_off = c * jnp.int32(half)

    def zrow(r, carry):
      for j in range(8):
        hist[pl.ds(r * (8 * LANES) + j * LANES, LANES)] = jnp.zeros(
            (LANES,), jnp.float32)
      return carry
    lax.fori_loop(jnp.int32(0), jnp.int32(h640 * W128 // (8 * LANES)), zrow,
                  jnp.int32(0))

    base_blk = s * jnp.int32(bpt)
    spread = lax.iota(jnp.int32, LANES)
    ones16 = jnp.full((LANES,), 1.0, jnp.float32)

    def wait_idx(b):
      pltpu.make_async_copy(edges_hbm.at[base_blk], ed[b], isem[b]).wait()

    pltpu.async_copy(edges_hbm.at[base_blk], ed[0], isem[0])

    def body(g, carry):
      for b in range(NBUF):
        i = NBUF * g + jnp.int32(b)
        nb = (b + 1) % NBUF

        wait_idx(b)
        for j in range(CH // LANES):
          loc = ed[b][pl.ds(CH + j * LANES, LANES)] - c_off
          ok = (loc >= 0) & (loc < half)
          trash = jnp.int32(half) + (spread & 7)
          loc = jnp.where(ok, loc, trash)
          flat = loc * LANES + spread
          plsc.addupdate_scatter(hist, (flat,), ones16)
        if b == NBUF - 1:
          @pl.when(g < outer - 1)
          def _():
            pltpu.async_copy(edges_hbm.at[base_blk + i + 1], ed[nb], isem[nb])
        else:
          pltpu.async_copy(edges_hbm.at[base_blk + i + 1], ed[nb], isem[nb])
      return carry

    lax.fori_loop(jnp.int32(0), jnp.int32(outer), body, jnp.int32(0))
    pltpu.sync_copy(hist.at[pl.ds(0, out_rows * W128)], out_hbm.at[c, s])

  return degk


def _tc_layer(n, d, blk, residual):
  grid = (n // blk,)
  half = n // NC
  blocks_per_core = half // blk
  assert half % blk == 0

  def body(x_ref, alo_ref, ahi_ref, dg_ref, ws_ref, wlo_ref, whi_ref, b_ref,
           o_ref, olo_ref, ohi_ref):
    x = x_ref[...]
    deg = jnp.sum(dg_ref[0], axis=(0, 2))[:, None]
    invd = 1.0 / jnp.maximum(deg, 1.0)
    h = (
        jnp.dot(x, ws_ref[...], preferred_element_type=jnp.float32)
        + jnp.dot(alo_ref[...] * invd, wlo_ref[...],
                  preferred_element_type=jnp.float32)
        + jnp.dot(ahi_ref[...] * invd, whi_ref[...],
                  preferred_element_type=jnp.float32)
        + b_ref[...]
    )
    h = jnp.maximum(h, 0.0)
    o_ref[...] = x + h if residual else h
    olo_ref[...] = h[:, :W128]
    ohi_ref[...] = h[:, W128:]

  z = np.int32(0)
  row_map = lambda i: (i, z)
  const_map = lambda i: (z, z)
  deg_map = lambda i: (i // blocks_per_core, z, i % blocks_per_core, z)
  return pl.pallas_call(
      body,
      grid=grid,
      in_specs=[
          pl.BlockSpec((blk, d), row_map),
          pl.BlockSpec((blk, W128), row_map),
          pl.BlockSpec((blk, W128), row_map),
          pl.BlockSpec((1, NS, blk, LANES), deg_map),
          pl.BlockSpec((d, d), const_map),
          pl.BlockSpec((W128, d), const_map),
          pl.BlockSpec((W128, d), const_map),
          pl.BlockSpec((1, d), const_map),
      ],
      out_specs=(pl.BlockSpec((blk, d), row_map),
                 pl.BlockSpec((blk, W128), row_map),
                 pl.BlockSpec((blk, W128), row_map)),
      out_shape=(jax.ShapeDtypeStruct((n, d), jnp.float32),
                 jax.ShapeDtypeStruct((n, W128), jnp.float32),
                 jax.ShapeDtypeStruct((n, W128), jnp.float32)),
  )


@jax.jit
def kernel(X, edge_index, W1_self, W1_neigh, b1, W2_self, W2_neigh, b2):
  n, d = X.shape
  e = edge_index.shape[1]

  src = edge_index[0].astype(jnp.int32)
  dst = edge_index[1].astype(jnp.int32)
  epc = NS * CH * NBUF
  e_pad = ((e + epc - 1) // epc) * epc
  pad = e_pad - e
  if pad:
    arp = jnp.arange(pad, dtype=jnp.int32)
    src = jnp.concatenate([src, (arp * 37) % n])
    dst = jnp.concatenate([dst, n + (arp & 7)])
  n_blocks = e_pad // CH
  edges = (jnp.stack([src, dst]).reshape(2, n_blocks, CH)
           .transpose(1, 0, 2).reshape(n_blocks, 2 * CH))

  seg = _seg_sum_builder(n, n_blocks)
  degk = _deg_builder(n, n_blocks)
  tc1 = _tc_layer(n, d, 1000, residual=False)
  tc2 = _tc_layer(n, d, 1000, residual=True)

  f32 = jnp.float32
  w1s, w1n = W1_self.astype(f32), W1_neigh.astype(f32)
  w2s, w2n = W2_self.astype(f32), W2_neigh.astype(f32)
  bb1 = b1.astype(f32).reshape(1, d)
  bb2 = b2.astype(f32).reshape(1, d)
  w1n_lo = lax.slice(w1n, (0, 0), (W128, d))
  w1n_hi = lax.slice(w1n, (W128, 0), (d, d))
  w2n_lo = lax.slice(w2n, (0, 0), (W128, d))
  w2n_hi = lax.slice(w2n, (W128, 0), (d, d))

  Xf = X.astype(f32)
  x_lo = lax.slice(Xf, (0, 0), (n, W128))
  x_hi = lax.slice(Xf, (0, W128), (n, d))
  half = n // NC
  deg = degk(edges).reshape(NC, NS, half, LANES)
  a1_lo, a1_hi = seg(x_lo, x_hi, edges)
  X1, x1_lo, x1_hi = tc1(Xf, a1_lo, a1_hi, deg, w1s, w1n_lo, w1n_hi, bb1)
  a2_lo, a2_hi = seg(x1_lo, x1_hi, edges)
  out, _, _ = tc2(X1, a2_lo, a2_hi, deg, w2s, w2n_lo, w2n_hi, bb2)
  out_dtype = jnp.result_type(X.dtype, W1_self.dtype)
  return out.astype(out_dtype)


# speedup vs baseline: 1.0134x; 1.0043x over previous
import functools

import numpy as np

import jax
import jax.numpy as jnp
from jax import lax
from jax.experimental import pallas as pl
from jax.experimental.pallas import tpu as pltpu
from jax.experimental.pallas import tpu_sc as plsc

NC = 2
NS = 16
LANES = 16
CH = 128
W128 = 128
ZR = 64
NBUF = 2


def _splits(n):
  half = n // NC
  acc_rows = half + 8
  rows_lo = (half // (8 * NS)) * 8
  rows_last = half - (NS - 1) * rows_lo
  assert half % 8 == 0 and rows_last % 8 == 0 and rows_last > 0
  return half, acc_rows, rows_lo, rows_last


def _remap_dst(ed_v, dst_v, c_off, half, spread):
  for j in range(CH // LANES):
    loc = ed_v[pl.ds(CH + j * LANES, LANES)] - c_off
    ok = (loc >= 0) & (loc < half)
    trash = jnp.int32(half) + (spread & 7)
    dst_v[pl.ds(j * LANES, LANES)] = jnp.where(ok, loc, trash)


def _copy_out(acc_sh, out_hbm, s, c_off, rows_lo, rows_last):
  @pl.when(s < NS - 1)
  def _():
    r0 = s * rows_lo
    pltpu.sync_copy(acc_sh.at[pl.ds(r0, rows_lo)],
                    out_hbm.at[pl.ds(c_off + r0, rows_lo)])

  @pl.when(s == NS - 1)
  def _():
    r0 = (NS - 1) * rows_lo
    pltpu.sync_copy(acc_sh.at[pl.ds(r0, rows_last)],
                    out_hbm.at[pl.ds(c_off + r0, rows_last)])


def _zero_acc(buf, accs, s, rows_lo, rows_last):
  d = buf.shape[1]

  def zrow(r, carry):
    for j in range(d // LANES):
      buf[r, pl.ds(j * LANES, LANES)] = jnp.zeros((LANES,), jnp.float32)
    return carry
  lax.fori_loop(jnp.int32(0), jnp.int32(ZR), zrow, jnp.int32(0))

  def zspan(r0, nrows):
    off = 0
    while off < nrows:
      step = min(ZR, nrows - off)
      for acc in accs:
        pltpu.sync_copy(buf.at[pl.ds(0, step)], acc.at[pl.ds(r0 + off, step)])
      off += step

  @pl.when(s < NS - 1)
  def _():
    zspan(s * rows_lo, rows_lo)

  @pl.when(s == NS - 1)
  def _():
    zspan((NS - 1) * rows_lo, rows_last + 8)


def _seg_sum_builder(n, n_blocks):
  acc_rows = n + 8
  rows_lo = (n // (8 * NS)) * 8
  rows_last = n - (NS - 1) * rows_lo
  assert n % 8 == 0 and rows_last % 8 == 0 and rows_last > 0
  bpt = n_blocks // NS
  assert bpt % NBUF == 0
  outer = bpt // NBUF

  mesh = plsc.VectorSubcoreMesh(core_axis_name="c", subcore_axis_name="s")

  @functools.partial(
      pl.kernel,
      out_type=(jax.ShapeDtypeStruct((n, W128), jnp.float32),
                jax.ShapeDtypeStruct((n, W128), jnp.float32)),
      mesh=mesh,
      scratch_types=(
          [pltpu.VMEM((2 * CH,), jnp.int32) for _ in range(NBUF)]
          + [pltpu.VMEM((CH,), jnp.int32) for _ in range(NBUF)]
          + [pltpu.VMEM((CH, W128), jnp.float32) for _ in range(NBUF)]
          + [pltpu.VMEM_SHARED((acc_rows, W128), jnp.float32)]
          + [pltpu.SemaphoreType.DMA for _ in range(3 * NBUF)]
      ),
  )
  def seg_sum(x_lo_hbm, x_hi_hbm, edges_hbm, out_lo_hbm, out_hi_hbm, *scr):
    ed = scr[0:NBUF]
    dst = scr[NBUF:2 * NBUF]
    ra = scr[2 * NBUF:3 * NBUF]
    acc = scr[3 * NBUF]
    isem = scr[3 * NBUF + 1:3 * NBUF + 1 + NBUF]
    gsem = scr[3 * NBUF + 1 + NBUF:3 * NBUF + 1 + 2 * NBUF]
    ssem = scr[3 * NBUF + 1 + 2 * NBUF:3 * NBUF + 1 + 3 * NBUF]

    c = lax.axis_index("c")
    s = lax.axis_index("s")

    _zero_acc(ra[0], (acc,), s, rows_lo, rows_last)
    plsc.subcore_barrier()

    base_blk = s * jnp.int32(bpt)

    def copy_dst(b):
      for j in range(CH // LANES):
        dst[b][pl.ds(j * LANES, LANES)] = ed[b][pl.ds(CH + j * LANES, LANES)]

    def gather(b):
      src = ed[b].at[pl.ds(0, CH)]

      @pl.when(c == 0)
      def _():
        pltpu.async_copy(x_lo_hbm.at[src], ra[b], gsem[b])

      @pl.when(c == 1)
      def _():
        pltpu.async_copy(x_hi_hbm.at[src], ra[b], gsem[b])

    def wait_gather(b):
      src = ed[b].at[pl.ds(0, CH)]
      pltpu.make_async_copy(x_lo_hbm.at[src], ra[b], gsem[b]).wait()

    def scatter(b):
      pltpu.async_copy(ra[b], acc.at[dst[b]], ssem[b], add=True)

    def wait_scatter(b):
      pltpu.make_async_copy(ra[b], acc.at[dst[b]], ssem[b]).wait()

    def wait_idx(b):
      pltpu.make_async_copy(edges_hbm.at[base_blk], ed[b], isem[b]).wait()

    pltpu.async_copy(edges_hbm.at[base_blk], ed[0], isem[0])

    def body(g, carry):
      for b in range(NBUF):
        i = NBUF * g + jnp.int32(b)
        pb = (b - 1) % NBUF
        nb = (b + 1) % NBUF

        @pl.when(g > 0)
        def _():
          wait_scatter(b)
        wait_idx(b)
        copy_dst(b)
        gather(b)
        if b == 0:
          @pl.when(g > 0)
          def _():
            wait_gather(pb)
            scatter(pb)
        else:
          wait_gather(pb)
          scatter(pb)
        if b == NBUF - 1:
          @pl.when(g < outer - 1)
          def _():
            pltpu.async_copy(edges_hbm.at[base_blk + i + 1], ed[nb], isem[nb])
        else:
          pltpu.async_copy(edges_hbm.at[base_blk + i + 1], ed[nb], isem[nb])
      return carry

    lax.fori_loop(jnp.int32(0), jnp.int32(outer), body, jnp.int32(0))

    last = NBUF - 1
    wait_gather(last)
    scatter(last)
    for b in range(NBUF):
      wait_scatter(b)

    plsc.subcore_barrier()

    @pl.when(c == 0)
    def _():
      _copy_out(acc, out_lo_hbm, s, jnp.int32(0), rows_lo, rows_last)

    @pl.when(c == 1)
    def _():
      _copy_out(acc, out_hi_hbm, s, jnp.int32(0), rows_lo, rows_last)

  return seg_sum


def _deg_builder(n, n_blocks):
  half, acc_rows, _, _ = _splits(n)
  bpt = n_blocks // NS
  assert bpt % NBUF == 0
  outer = bpt // NBUF
  h640 = acc_rows * LANES // W128
  out_rows = half * LANES // W128
  assert half * LANES % W128 == 0

  mesh = plsc.VectorSubcoreMesh(core_axis_name="c", subcore_axis_name="s")

  @functools.partial(
      pl.kernel,
      out_type=jax.ShapeDtypeStruct((NC, NS, out_rows * W128), jnp.float32),
      mesh=mesh,
      compiler_params=pltpu.CompilerParams(needs_layout_passes=False),
      scratch_types=(
          [pltpu.VMEM((2 * CH,), jnp.int32) for _ in range(NBUF)]
          + [pltpu.VMEM((h640 * W128,), jnp.float32)]
          + [pltpu.SemaphoreType.DMA for _ in range(NBUF)]
      ),
  )
  def degk(edges_hbm, out_hbm, *scr):
    ed = scr[0:NBUF]
    hist = scr[NBUF]
    isem = scr[NBUF + 1:NBUF + 1 + NBUF]

    c = lax.axis_index("c")
    s = lax.axis_index("s")
    c_off = c * jnp.int32(half)

    def zrow(r, carry):
      for j in range(8):
        hist[pl.ds(r * (8 * LANES) + j * LANES, LANES)] = jnp.zeros(
            (LANES,), jnp.float32)
      return carry
    lax.fori_loop(jnp.int32(0), jnp.int32(h640 * W128 // (8 * LANES)), zrow,
                  jnp.int32(0))

    base_blk = s * jnp.int32(bpt)
    spread = lax.iota(jnp.int32, LANES)
    ones16 = jnp.full((LANES,), 1.0, jnp.float32)

    def wait_idx(b):
      pltpu.make_async_copy(edges_hbm.at[base_blk], ed[b], isem[b]).wait()

    pltpu.async_copy(edges_hbm.at[base_blk], ed[0], isem[0])

    def body(g, carry):
      for b in range(NBUF):
        i = NBUF * g + jnp.int32(b)
        nb = (b + 1) % NBUF

        wait_idx(b)
        for j in range(CH // LANES):
          loc = ed[b][pl.ds(CH + j * LANES, LANES)] - c_off
          ok = (loc >= 0) & (loc < half)
          trash = jnp.int32(half) + (spread & 7)
          loc = jnp.where(ok, loc, trash)
          flat = loc * LANES + spread
          plsc.addupdate_scatter(hist, (flat,), ones16)
        if b == NBUF - 1:
          @pl.when(g < outer - 1)
          def _():
            pltpu.async_copy(edges_hbm.at[base_blk + i + 1], ed[nb], isem[nb])
        else:
          pltpu.async_copy(edges_hbm.at[base_blk + i + 1], ed[nb], isem[nb])
      return carry

    lax.fori_loop(jnp.int32(0), jnp.int32(outer), body, jnp.int32(0))
    pltpu.sync_copy(hist.at[pl.ds(0, out_rows * W128)], out_hbm.at[c, s])

  return degk


def _tc_layer(n, d, blk, residual):
  grid = (n // blk,)
  half = n // NC
  blocks_per_core = half // blk
  assert half % blk == 0

  def body(x_ref, alo_ref, ahi_ref, dg_ref, ws_ref, wlo_ref, whi_ref, b_ref,
           o_ref, *half_refs):
    x = x_ref[...]
    deg = jnp.sum(dg_ref[0], axis=(0, 2))[:, None]
    invd = 1.0 / jnp.maximum(deg, 1.0)
    h = (
        jnp.dot(x, ws_ref[...], preferred_element_type=jnp.float32)
        + jnp.dot(alo_ref[...] * invd, wlo_ref[...],
                  preferred_element_type=jnp.float32)
        + jnp.dot(ahi_ref[...] * invd, whi_ref[...],
                  preferred_element_type=jnp.float32)
        + b_ref[...]
    )
    h = jnp.maximum(h, 0.0)
    o_ref[...] = x + h if residual else h
    if half_refs:
      olo_ref, ohi_ref = half_refs
      olo_ref[...] = h[:, :W128]
      ohi_ref[...] = h[:, W128:]

  z = np.int32(0)
  row_map = lambda i: (i, z)
  const_map = lambda i: (z, z)
  deg_map = lambda i: (i // blocks_per_core, z, i % blocks_per_core, z)
  out_specs = (pl.BlockSpec((blk, d), row_map),)
  out_shape = (jax.ShapeDtypeStruct((n, d), jnp.float32),)
  if not residual:
    out_specs += (pl.BlockSpec((blk, W128), row_map),
                  pl.BlockSpec((blk, W128), row_map))
    out_shape += (jax.ShapeDtypeStruct((n, W128), jnp.float32),
                  jax.ShapeDtypeStruct((n, W128), jnp.float32))
  return pl.pallas_call(
      body,
      grid=grid,
      in_specs=[
          pl.BlockSpec((blk, d), row_map),
          pl.BlockSpec((blk, W128), row_map),
          pl.BlockSpec((blk, W128), row_map),
          pl.BlockSpec((1, NS, blk, LANES), deg_map),
          pl.BlockSpec((d, d), const_map),
          pl.BlockSpec((W128, d), const_map),
          pl.BlockSpec((W128, d), const_map),
          pl.BlockSpec((1, d), const_map),
      ],
      out_specs=out_specs,
      out_shape=out_shape,
  )


@jax.jit
def kernel(X, edge_index, W1_self, W1_neigh, b1, W2_self, W2_neigh, b2):
  n, d = X.shape
  e = edge_index.shape[1]

  src = edge_index[0].astype(jnp.int32)
  dst = edge_index[1].astype(jnp.int32)
  epc = NS * CH * NBUF
  e_pad = ((e + epc - 1) // epc) * epc
  pad = e_pad - e
  if pad:
    arp = jnp.arange(pad, dtype=jnp.int32)
    src = jnp.concatenate([src, (arp * 37) % n])
    dst = jnp.concatenate([dst, n + (arp & 7)])
  n_blocks = e_pad // CH
  edges = (jnp.stack([src, dst]).reshape(2, n_blocks, CH)
           .transpose(1, 0, 2).reshape(n_blocks, 2 * CH))

  seg = _seg_sum_builder(n, n_blocks)
  degk = _deg_builder(n, n_blocks)
  tc1 = _tc_layer(n, d, 1000, residual=False)
  tc2 = _tc_layer(n, d, 1000, residual=True)

  f32 = jnp.float32
  w1s, w1n = W1_self.astype(f32), W1_neigh.astype(f32)
  w2s, w2n = W2_self.astype(f32), W2_neigh.astype(f32)
  bb1 = b1.astype(f32).reshape(1, d)
  bb2 = b2.astype(f32).reshape(1, d)
  w1n_lo = lax.slice(w1n, (0, 0), (W128, d))
  w1n_hi = lax.slice(w1n, (W128, 0), (d, d))
  w2n_lo = lax.slice(w2n, (0, 0), (W128, d))
  w2n_hi = lax.slice(w2n, (W128, 0), (d, d))

  Xf = X.astype(f32)
  x_lo = lax.slice(Xf, (0, 0), (n, W128))
  x_hi = lax.slice(Xf, (0, W128), (n, d))
  half = n // NC
  deg = degk(edges).reshape(NC, NS, half, LANES)
  a1_lo, a1_hi = seg(x_lo, x_hi, edges)
  X1, x1_lo, x1_hi = tc1(Xf, a1_lo, a1_hi, deg, w1s, w1n_lo, w1n_hi, bb1)
  a2_lo, a2_hi = seg(x1_lo, x1_hi, edges)
  (out,) = tc2(X1, a2_lo, a2_hi, deg, w2s, w2n_lo, w2n_hi, bb2)
  out_dtype = jnp.result_type(X.dtype, W1_self.dtype)
  return out.astype(out_dtype)
